# baseline (speedup 1.0000x reference)
import functools

import jax
import jax.numpy as jnp
from jax import lax
from jax.experimental import pallas as pl
from jax.experimental.pallas import tpu as pltpu

NUM_EXPERTS = 64
TOP_K = 8
E_BLK = 2
NBUF = 4


def _moe_body(hs_ref, gw_ref, gp_hbm, up_hbm, dp_hbm, out_ref, logits_ref,
              gp_buf, up_buf, dp_buf, comb_ref, sems):
    T, H = hs_ref.shape
    E = gw_ref.shape[0]
    I = gp_hbm.shape[1]
    n_steps = E // E_BLK

    def copies(e, slot):
        return (
            pltpu.make_async_copy(
                gp_hbm.at[pl.ds(e * E_BLK, E_BLK)], gp_buf.at[slot],
                sems.at[slot, 0]),
            pltpu.make_async_copy(
                up_hbm.at[pl.ds(e * E_BLK, E_BLK)], up_buf.at[slot],
                sems.at[slot, 1]),
            pltpu.make_async_copy(
                dp_hbm.at[pl.ds(e * E_BLK, E_BLK)], dp_buf.at[slot],
                sems.at[slot, 2]),
        )

    for s in range(NBUF):
        for c in copies(s, s):
            c.start()

    hs = hs_ref[...]
    logits = jax.lax.dot_general(
        hs, gw_ref[...], (((1,), (1,)), ((), ())),
        preferred_element_type=jnp.float32)
    logits_ref[...] = logits
    probs = jax.nn.softmax(logits, axis=1)
    colid = jax.lax.broadcasted_iota(jnp.int32, (T, E), 1)
    comb = jnp.zeros_like(probs)
    p = probs
    for _ in range(TOP_K):
        m = jnp.max(p, axis=1, keepdims=True)
        idx = jnp.where(p == m, colid, E)
        sel = colid == jnp.min(idx, axis=1, keepdims=True)
        comb = jnp.where(sel, p, comb)
        p = jnp.where(sel, -1.0, p)
    comb = comb / jnp.sum(comb, axis=1, keepdims=True)
    comb_ref[...] = comb
    out_ref[...] = jnp.zeros_like(out_ref)

    def step(e, carry):
        slot = lax.rem(e, NBUF)
        for c in copies(e, slot):
            c.wait()

        gp = gp_buf[slot].reshape(E_BLK * I, H)
        up = up_buf[slot].reshape(E_BLK * I, H)
        g = jax.lax.dot_general(hs, gp, (((1,), (1,)), ((), ())),
                                preferred_element_type=jnp.float32)
        u = jax.lax.dot_general(hs, up, (((1,), (1,)), ((), ())),
                                preferred_element_type=jnp.float32)
        a = g * jax.nn.sigmoid(g) * u

        cmb = comb_ref[...]
        acc = out_ref[...]
        for j in range(E_BLK):
            ej = e * E_BLK + j
            w = jnp.sum(jnp.where(colid == ej, cmb, 0.0), axis=1,
                        keepdims=True)
            aw = a[:, j * I:(j + 1) * I] * w
            acc = acc + jax.lax.dot_general(
                aw, dp_buf[slot, j], (((1,), (1,)), ((), ())),
                preferred_element_type=jnp.float32)
        out_ref[...] = acc

        @pl.when(e + NBUF < n_steps)
        def _():
            for c in copies(e + NBUF, slot):
                c.start()

        return carry

    lax.fori_loop(0, n_steps, step, 0)


@functools.partial(jax.jit, static_argnames=())
def kernel(hidden_states, gate_w, gate_proj, up_proj, down_proj):
    B, S, H = hidden_states.shape
    T = B * S
    hs = hidden_states.reshape(T, H)
    E = gate_w.shape[0]
    I = gate_proj.shape[1]

    final, logits = pl.pallas_call(
        _moe_body,
        in_specs=[
            pl.BlockSpec(memory_space=pltpu.VMEM),
            pl.BlockSpec(memory_space=pltpu.VMEM),
            pl.BlockSpec(memory_space=pl.ANY),
            pl.BlockSpec(memory_space=pl.ANY),
            pl.BlockSpec(memory_space=pl.ANY),
        ],
        out_specs=[
            pl.BlockSpec(memory_space=pltpu.VMEM),
            pl.BlockSpec(memory_space=pltpu.VMEM),
        ],
        out_shape=[
            jax.ShapeDtypeStruct((T, H), jnp.float32),
            jax.ShapeDtypeStruct((T, E), jnp.float32),
        ],
        scratch_shapes=[
            pltpu.VMEM((NBUF, E_BLK, I, H), jnp.float32),
            pltpu.VMEM((NBUF, E_BLK, I, H), jnp.float32),
            pltpu.VMEM((NBUF, E_BLK, H, I), jnp.float32),
            pltpu.VMEM((T, E), jnp.float32),
            pltpu.SemaphoreType.DMA((NBUF, 3)),
        ],
    )(hs, gate_w, gate_proj, up_proj, down_proj)

    return final.reshape(B, S, H), logits


# speedup vs baseline: 1.0071x; 1.0071x over previous
import functools

import jax
import jax.numpy as jnp
from jax import lax
from jax.experimental import pallas as pl
from jax.experimental.pallas import tpu as pltpu

NUM_EXPERTS = 64
TOP_K = 8
E_BLK = 2
NBUF = 3


def _moe_body(hs_ref, gw_ref, gp_hbm, up_hbm, dp_hbm, out_ref, logits_ref,
              gp_buf, up_buf, dp_buf, comb_ref, sems):
    T, H = hs_ref.shape
    E = gw_ref.shape[0]
    I = gp_hbm.shape[1]
    n_steps = E // E_BLK

    def copies(e, slot):
        Ih = I // 2
        Hh = H // 2
        return (
            pltpu.make_async_copy(
                gp_hbm.at[pl.ds(e * E_BLK, E_BLK), pl.ds(0, Ih)],
                gp_buf.at[slot, :, pl.ds(0, Ih)], sems.at[slot, 0]),
            pltpu.make_async_copy(
                gp_hbm.at[pl.ds(e * E_BLK, E_BLK), pl.ds(Ih, Ih)],
                gp_buf.at[slot, :, pl.ds(Ih, Ih)], sems.at[slot, 1]),
            pltpu.make_async_copy(
                up_hbm.at[pl.ds(e * E_BLK, E_BLK), pl.ds(0, Ih)],
                up_buf.at[slot, :, pl.ds(0, Ih)], sems.at[slot, 2]),
            pltpu.make_async_copy(
                up_hbm.at[pl.ds(e * E_BLK, E_BLK), pl.ds(Ih, Ih)],
                up_buf.at[slot, :, pl.ds(Ih, Ih)], sems.at[slot, 3]),
            pltpu.make_async_copy(
                dp_hbm.at[pl.ds(e * E_BLK, E_BLK), pl.ds(0, Hh)],
                dp_buf.at[slot, :, pl.ds(0, Hh)], sems.at[slot, 4]),
            pltpu.make_async_copy(
                dp_hbm.at[pl.ds(e * E_BLK, E_BLK), pl.ds(Hh, Hh)],
                dp_buf.at[slot, :, pl.ds(Hh, Hh)], sems.at[slot, 5]),
        )

    for s in range(NBUF):
        for c in copies(s, s):
            c.start()

    hs = hs_ref[...]
    logits = jax.lax.dot_general(
        hs, gw_ref[...], (((1,), (1,)), ((), ())),
        preferred_element_type=jnp.float32)
    logits_ref[...] = logits
    probs = jax.nn.softmax(logits, axis=1)
    colid = jax.lax.broadcasted_iota(jnp.int32, (T, E), 1)
    comb = jnp.zeros_like(probs)
    p = probs
    for _ in range(TOP_K):
        m = jnp.max(p, axis=1, keepdims=True)
        idx = jnp.where(p == m, colid, E)
        sel = colid == jnp.min(idx, axis=1, keepdims=True)
        comb = jnp.where(sel, p, comb)
        p = jnp.where(sel, -1.0, p)
    comb = comb / jnp.sum(comb, axis=1, keepdims=True)
    comb_ref[...] = comb
    out_ref[...] = jnp.zeros_like(out_ref)

    def step(e, carry):
        slot = lax.rem(e, NBUF)
        for c in copies(e, slot):
            c.wait()

        gp = gp_buf[slot].reshape(E_BLK * I, H)
        up = up_buf[slot].reshape(E_BLK * I, H)
        g = jax.lax.dot_general(hs, gp, (((1,), (1,)), ((), ())),
                                preferred_element_type=jnp.float32)
        u = jax.lax.dot_general(hs, up, (((1,), (1,)), ((), ())),
                                preferred_element_type=jnp.float32)
        a = g * jax.nn.sigmoid(g) * u

        cmb = comb_ref[...]
        acc = out_ref[...]
        for j in range(E_BLK):
            ej = e * E_BLK + j
            w = jnp.sum(jnp.where(colid == ej, cmb, 0.0), axis=1,
                        keepdims=True)
            aw = a[:, j * I:(j + 1) * I] * w
            acc = acc + jax.lax.dot_general(
                aw, dp_buf[slot, j], (((1,), (1,)), ((), ())),
                preferred_element_type=jnp.float32)
        out_ref[...] = acc

        @pl.when(e + NBUF < n_steps)
        def _():
            for c in copies(e + NBUF, slot):
                c.start()

        return carry

    lax.fori_loop(0, n_steps, step, 0)


@functools.partial(jax.jit, static_argnames=())
def kernel(hidden_states, gate_w, gate_proj, up_proj, down_proj):
    B, S, H = hidden_states.shape
    T = B * S
    hs = hidden_states.reshape(T, H)
    E = gate_w.shape[0]
    I = gate_proj.shape[1]

    final, logits = pl.pallas_call(
        _moe_body,
        in_specs=[
            pl.BlockSpec(memory_space=pltpu.VMEM),
            pl.BlockSpec(memory_space=pltpu.VMEM),
            pl.BlockSpec(memory_space=pl.ANY),
            pl.BlockSpec(memory_space=pl.ANY),
            pl.BlockSpec(memory_space=pl.ANY),
        ],
        out_specs=[
            pl.BlockSpec(memory_space=pltpu.VMEM),
            pl.BlockSpec(memory_space=pltpu.VMEM),
        ],
        out_shape=[
            jax.ShapeDtypeStruct((T, H), jnp.float32),
            jax.ShapeDtypeStruct((T, E), jnp.float32),
        ],
        scratch_shapes=[
            pltpu.VMEM((NBUF, E_BLK, I, H), jnp.float32),
            pltpu.VMEM((NBUF, E_BLK, I, H), jnp.float32),
            pltpu.VMEM((NBUF, E_BLK, H, I), jnp.float32),
            pltpu.VMEM((T, E), jnp.float32),
            pltpu.SemaphoreType.DMA((NBUF, 6)),
        ],
    )(hs, gate_w, gate_proj, up_proj, down_proj)

    return final.reshape(B, S, H), logits


# speedup vs baseline: 1.0091x; 1.0020x over previous
import functools

import jax
import jax.numpy as jnp
from jax import lax
from jax.experimental import pallas as pl
from jax.experimental.pallas import tpu as pltpu

NUM_EXPERTS = 64
TOP_K = 8
E_BLK = 2
NBUF = 3


def _moe_body(hs_ref, gw_ref, gp_hbm, up_hbm, dp_hbm, out_ref, logits_ref,
              gp_buf, up_buf, dp_buf, comb_ref, sems):
    T, H = hs_ref.shape
    E = gw_ref.shape[0]
    I = gp_hbm.shape[1]
    n_steps = E // E_BLK

    def copies(e, slot):
        return (
            pltpu.make_async_copy(
                gp_hbm.at[pl.ds(e * E_BLK, E_BLK)], gp_buf.at[slot],
                sems.at[slot, 0]),
            pltpu.make_async_copy(
                up_hbm.at[pl.ds(e * E_BLK, E_BLK)], up_buf.at[slot],
                sems.at[slot, 1]),
            pltpu.make_async_copy(
                dp_hbm.at[pl.ds(e * E_BLK, E_BLK)], dp_buf.at[slot],
                sems.at[slot, 2]),
        )

    for s in range(NBUF):
        for c in copies(s, s):
            c.start()

    hs = hs_ref[...]
    logits = jax.lax.dot_general(
        hs, gw_ref[...], (((1,), (1,)), ((), ())),
        preferred_element_type=jnp.float32)
    logits_ref[...] = logits
    probs = jax.nn.softmax(logits, axis=1)
    colid = jax.lax.broadcasted_iota(jnp.int32, (T, E), 1)
    comb = jnp.zeros_like(probs)
    p = probs
    for _ in range(TOP_K):
        m = jnp.max(p, axis=1, keepdims=True)
        idx = jnp.where(p == m, colid, E)
        sel = colid == jnp.min(idx, axis=1, keepdims=True)
        comb = jnp.where(sel, p, comb)
        p = jnp.where(sel, -1.0, p)
    comb = comb / jnp.sum(comb, axis=1, keepdims=True)
    comb_ref[...] = comb
    out_ref[...] = jnp.zeros_like(out_ref)

    def step(e, carry):
        slot = lax.rem(e, NBUF)
        for c in copies(e, slot):
            c.wait()

        gp = gp_buf[slot].reshape(E_BLK * I, H)
        up = up_buf[slot].reshape(E_BLK * I, H)
        g = jax.lax.dot_general(hs, gp, (((1,), (1,)), ((), ())),
                                preferred_element_type=jnp.float32)
        u = jax.lax.dot_general(hs, up, (((1,), (1,)), ((), ())),
                                preferred_element_type=jnp.float32)
        a = g * jax.nn.sigmoid(g) * u

        cmb = comb_ref[...]
        acc = out_ref[...]
        for j in range(E_BLK):
            ej = e * E_BLK + j
            w = jnp.sum(jnp.where(colid == ej, cmb, 0.0), axis=1,
                        keepdims=True)
            aw = a[:, j * I:(j + 1) * I] * w
            acc = acc + jax.lax.dot_general(
                aw, dp_buf[slot, j], (((1,), (1,)), ((), ())),
                preferred_element_type=jnp.float32)
        out_ref[...] = acc

        @pl.when(e + NBUF < n_steps)
        def _():
            for c in copies(e + NBUF, slot):
                c.start()

        return carry

    lax.fori_loop(0, n_steps, step, 0)


@functools.partial(jax.jit, static_argnames=())
def kernel(hidden_states, gate_w, gate_proj, up_proj, down_proj):
    B, S, H = hidden_states.shape
    T = B * S
    hs = hidden_states.reshape(T, H)
    E = gate_w.shape[0]
    I = gate_proj.shape[1]

    final, logits = pl.pallas_call(
        _moe_body,
        in_specs=[
            pl.BlockSpec(memory_space=pltpu.VMEM),
            pl.BlockSpec(memory_space=pltpu.VMEM),
            pl.BlockSpec(memory_space=pl.ANY),
            pl.BlockSpec(memory_space=pl.ANY),
            pl.BlockSpec(memory_space=pl.ANY),
        ],
        out_specs=[
            pl.BlockSpec(memory_space=pltpu.VMEM),
            pl.BlockSpec(memory_space=pltpu.VMEM),
        ],
        out_shape=[
            jax.ShapeDtypeStruct((T, H), jnp.float32),
            jax.ShapeDtypeStruct((T, E), jnp.float32),
        ],
        scratch_shapes=[
            pltpu.VMEM((NBUF, E_BLK, I, H), jnp.float32),
            pltpu.VMEM((NBUF, E_BLK, I, H), jnp.float32),
            pltpu.VMEM((NBUF, E_BLK, H, I), jnp.float32),
            pltpu.VMEM((T, E), jnp.float32),
            pltpu.SemaphoreType.DMA((NBUF, 3)),
        ],
    )(hs, gate_w, gate_proj, up_proj, down_proj)

    return final.reshape(B, S, H), logits


# speedup vs baseline: 1.0128x; 1.0036x over previous
import functools

import jax
import jax.numpy as jnp
from jax import lax
from jax.experimental import pallas as pl
from jax.experimental.pallas import tpu as pltpu

NUM_EXPERTS = 64
TOP_K = 8
E_BLK = 1
NBUF = 6


def _moe_body(hs_ref, gw_ref, gp_hbm, up_hbm, dp_hbm, out_ref, logits_ref,
              gp_buf, up_buf, dp_buf, comb_ref, sems):
    T, H = hs_ref.shape
    E = gw_ref.shape[0]
    I = gp_hbm.shape[1]
    n_steps = E // E_BLK

    def copies(e, slot):
        return (
            pltpu.make_async_copy(
                gp_hbm.at[pl.ds(e * E_BLK, E_BLK)], gp_buf.at[slot],
                sems.at[slot, 0]),
            pltpu.make_async_copy(
                up_hbm.at[pl.ds(e * E_BLK, E_BLK)], up_buf.at[slot],
                sems.at[slot, 1]),
            pltpu.make_async_copy(
                dp_hbm.at[pl.ds(e * E_BLK, E_BLK)], dp_buf.at[slot],
                sems.at[slot, 2]),
        )

    for s in range(NBUF):
        for c in copies(s, s):
            c.start()

    hs = hs_ref[...]
    logits = jax.lax.dot_general(
        hs, gw_ref[...], (((1,), (1,)), ((), ())),
        preferred_element_type=jnp.float32)
    logits_ref[...] = logits
    probs = jax.nn.softmax(logits, axis=1)
    colid = jax.lax.broadcasted_iota(jnp.int32, (T, E), 1)
    comb = jnp.zeros_like(probs)
    p = probs
    for _ in range(TOP_K):
        m = jnp.max(p, axis=1, keepdims=True)
        idx = jnp.where(p == m, colid, E)
        sel = colid == jnp.min(idx, axis=1, keepdims=True)
        comb = jnp.where(sel, p, comb)
        p = jnp.where(sel, -1.0, p)
    comb = comb / jnp.sum(comb, axis=1, keepdims=True)
    comb_ref[...] = comb
    out_ref[...] = jnp.zeros_like(out_ref)

    def step(e, carry):
        slot = lax.rem(e, NBUF)
        for c in copies(e, slot):
            c.wait()

        gp = gp_buf[slot].reshape(E_BLK * I, H)
        up = up_buf[slot].reshape(E_BLK * I, H)
        g = jax.lax.dot_general(hs, gp, (((1,), (1,)), ((), ())),
                                preferred_element_type=jnp.float32)
        u = jax.lax.dot_general(hs, up, (((1,), (1,)), ((), ())),
                                preferred_element_type=jnp.float32)
        a = g * jax.nn.sigmoid(g) * u

        cmb = comb_ref[...]
        acc = out_ref[...]
        for j in range(E_BLK):
            ej = e * E_BLK + j
            w = jnp.sum(jnp.where(colid == ej, cmb, 0.0), axis=1,
                        keepdims=True)
            aw = a[:, j * I:(j + 1) * I] * w
            acc = acc + jax.lax.dot_general(
                aw, dp_buf[slot, j], (((1,), (1,)), ((), ())),
                preferred_element_type=jnp.float32)
        out_ref[...] = acc

        @pl.when(e + NBUF < n_steps)
        def _():
            for c in copies(e + NBUF, slot):
                c.start()

        return carry

    lax.fori_loop(0, n_steps, step, 0)


@functools.partial(jax.jit, static_argnames=())
def kernel(hidden_states, gate_w, gate_proj, up_proj, down_proj):
    B, S, H = hidden_states.shape
    T = B * S
    hs = hidden_states.reshape(T, H)
    E = gate_w.shape[0]
    I = gate_proj.shape[1]

    final, logits = pl.pallas_call(
        _moe_body,
        in_specs=[
            pl.BlockSpec(memory_space=pltpu.VMEM),
            pl.BlockSpec(memory_space=pltpu.VMEM),
            pl.BlockSpec(memory_space=pl.ANY),
            pl.BlockSpec(memory_space=pl.ANY),
            pl.BlockSpec(memory_space=pl.ANY),
        ],
        out_specs=[
            pl.BlockSpec(memory_space=pltpu.VMEM),
            pl.BlockSpec(memory_space=pltpu.VMEM),
        ],
        out_shape=[
            jax.ShapeDtypeStruct((T, H), jnp.float32),
            jax.ShapeDtypeStruct((T, E), jnp.float32),
        ],
        scratch_shapes=[
            pltpu.VMEM((NBUF, E_BLK, I, H), jnp.float32),
            pltpu.VMEM((NBUF, E_BLK, I, H), jnp.float32),
            pltpu.VMEM((NBUF, E_BLK, H, I), jnp.float32),
            pltpu.VMEM((T, E), jnp.float32),
            pltpu.SemaphoreType.DMA((NBUF, 3)),
        ],
    )(hs, gate_w, gate_proj, up_proj, down_proj)

    return final.reshape(B, S, H), logits
